# baseline (speedup 1.0000x reference)
import functools

import jax
import jax.numpy as jnp
from jax import lax
from jax.experimental import pallas as pl
from jax.experimental.pallas import tpu as pltpu
from jax.experimental.pallas import tpu_sc as plsc

H = 256
F32 = jnp.float32


def _ln(h):
    m = jnp.mean(h, axis=-1, keepdims=True)
    c = h - m
    v = jnp.mean(c * c, axis=-1, keepdims=True)
    return c * lax.rsqrt(v + 1e-5)


def _dot(a, b):
    return jnp.dot(a, b, preferred_element_type=F32)


def _embed3_body(x_ref, w1, b1, w2, b2, w3, b3, o_ref):
    h = jnp.maximum(_dot(x_ref[...], w1[...]) + b1[...], 0.0)
    h = jnp.maximum(_dot(h, w2[...]) + b2[...], 0.0)
    h = _dot(h, w3[...]) + b3[...]
    o_ref[...] = _ln(h)


def _embed3(x, ps, bm):
    (w1, b1), (w2, b2), (w3, b3) = ps
    M, K = x.shape
    w1 = jnp.pad(w1, ((0, K - w1.shape[0]), (0, 0)))
    grid = M // bm
    wspec = lambda r, c: pl.BlockSpec((r, c), lambda i: (0, 0))
    return pl.pallas_call(
        _embed3_body,
        grid=(grid,),
        in_specs=[
            pl.BlockSpec((bm, K), lambda i: (i, 0)),
            wspec(K, H), wspec(1, H), wspec(H, H), wspec(1, H), wspec(H, H), wspec(1, H),
        ],
        out_specs=pl.BlockSpec((bm, H), lambda i: (i, 0)),
        out_shape=jax.ShapeDtypeStruct((M, H), F32),
        compiler_params=pltpu.CompilerParams(dimension_semantics=("arbitrary",)),
    )(x, w1, b1.reshape(1, H), w2, b2.reshape(1, H), w3, b3.reshape(1, H))


def _edge3_body(e_ref, gb_ref, gc_ref, w1a, b1, w2, b2, w3, b3, o_ref):
    e = e_ref[...]
    h = jnp.maximum(_dot(e, w1a[...]) + gb_ref[...] + gc_ref[...] + b1[...], 0.0)
    h = jnp.maximum(_dot(h, w2[...]) + b2[...], 0.0)
    h = _dot(h, w3[...]) + b3[...]
    o_ref[...] = e + _ln(h)


def _edge3(e, gb, gc, w1a, b1, w2, b2, w3, b3, bm):
    M = e.shape[0]
    grid = M // bm
    dspec = pl.BlockSpec((bm, H), lambda i: (i, 0))
    wspec = lambda r, c: pl.BlockSpec((r, c), lambda i: (0, 0))
    return pl.pallas_call(
        _edge3_body,
        grid=(grid,),
        in_specs=[dspec, dspec, dspec,
                  wspec(H, H), wspec(1, H), wspec(H, H), wspec(1, H), wspec(H, H), wspec(1, H)],
        out_specs=dspec,
        out_shape=jax.ShapeDtypeStruct((M, H), F32),
        compiler_params=pltpu.CompilerParams(dimension_semantics=("arbitrary",)),
    )(e, gb, gc, w1a, b1.reshape(1, H), w2, b2.reshape(1, H), w3, b3.reshape(1, H))


def _node3_body(nproj, x_ref, a_ref, v1a, v1b, b1, v2, b2, v3, b3, p1, p2, o_ref, pb_ref, pc_ref):
    x = x_ref[...]
    agg = a_ref[...]
    h = jnp.maximum(_dot(x, v1a[...]) + _dot(agg, v1b[...]) + b1[...], 0.0)
    h = jnp.maximum(_dot(h, v2[...]) + b2[...], 0.0)
    h = _dot(h, v3[...]) + b3[...]
    xn = x + _ln(h)
    o_ref[...] = xn
    if nproj:
        pb_ref[...] = _dot(xn, p1[...])
        pc_ref[...] = _dot(xn, p2[...])


def _node3_noproj_body(x_ref, a_ref, v1a, v1b, b1, v2, b2, v3, b3, o_ref):
    x = x_ref[...]
    agg = a_ref[...]
    h = jnp.maximum(_dot(x, v1a[...]) + _dot(agg, v1b[...]) + b1[...], 0.0)
    h = jnp.maximum(_dot(h, v2[...]) + b2[...], 0.0)
    h = _dot(h, v3[...]) + b3[...]
    o_ref[...] = x + _ln(h)


def _node3(x, a0, v1a, b1, v1b, v2, b2, v3, b3, proj, bm):
    M = x.shape[0]
    grid = M // bm
    dspec = pl.BlockSpec((bm, H), lambda i: (i, 0))
    wspec = lambda: pl.BlockSpec((H, H), lambda i: (0, 0))
    bspec = lambda: pl.BlockSpec((1, H), lambda i: (0, 0))
    if proj is None:
        return pl.pallas_call(
            _node3_noproj_body,
            grid=(grid,),
            in_specs=[dspec, dspec,
                      wspec(), wspec(), bspec(), wspec(), bspec(), wspec(), bspec()],
            out_specs=dspec,
            out_shape=jax.ShapeDtypeStruct((M, H), F32),
            compiler_params=pltpu.CompilerParams(dimension_semantics=("arbitrary",)),
        )(x, a0, v1a, v1b, b1.reshape(1, H), v2, b2.reshape(1, H), v3, b3.reshape(1, H))
    p1, p2 = proj
    return pl.pallas_call(
        functools.partial(_node3_body, True),
        grid=(grid,),
        in_specs=[dspec, dspec,
                  wspec(), wspec(), bspec(), wspec(), bspec(), wspec(), bspec(),
                  wspec(), wspec()],
        out_specs=(dspec, dspec, dspec),
        out_shape=(jax.ShapeDtypeStruct((M, H), F32),
                   jax.ShapeDtypeStruct((M, H), F32),
                   jax.ShapeDtypeStruct((M, H), F32)),
        compiler_params=pltpu.CompilerParams(dimension_semantics=("arbitrary",)),
    )(x, a0, v1a, v1b, b1.reshape(1, H), v2, b2.reshape(1, H), v3, b3.reshape(1, H), p1, p2)


def _proj_body(x_ref, w_ref, o_ref):
    o_ref[...] = _dot(x_ref[...], w_ref[...])


def _proj(x, w, bm):
    M = x.shape[0]
    N = w.shape[1]
    return pl.pallas_call(
        _proj_body,
        grid=(M // bm,),
        in_specs=[pl.BlockSpec((bm, H), lambda i: (i, 0)),
                  pl.BlockSpec((H, N), lambda i: (0, 0))],
        out_specs=pl.BlockSpec((bm, N), lambda i: (i, 0)),
        out_shape=jax.ShapeDtypeStruct((M, N), F32),
        compiler_params=pltpu.CompilerParams(dimension_semantics=("arbitrary",)),
    )(x, w)


def _out3_body(x_ref, w1, b1, w2, b2, w3, b3, o_ref):
    h = jnp.maximum(_dot(x_ref[...], w1[...]) + b1[...], 0.0)
    h = jnp.maximum(_dot(h, w2[...]) + b2[...], 0.0)
    o_ref[...] = _dot(h, w3[...]) + b3[...]


def _out3(x, ps):
    (w1, b1), (w2, b2), (w3, b3) = ps
    M = x.shape[0]
    N = 128
    w3p = jnp.pad(w3, ((0, 0), (0, N - w3.shape[1])))
    b3p = jnp.pad(b3, (0, N - b3.shape[0])).reshape(1, N)
    wspec = lambda r, c: pl.BlockSpec((r, c), lambda i: (0, 0))
    return pl.pallas_call(
        _out3_body,
        grid=(1,),
        in_specs=[pl.BlockSpec((M, H), lambda i: (0, 0)),
                  wspec(H, H), wspec(1, H), wspec(H, H), wspec(1, H), wspec(H, N), wspec(1, N)],
        out_specs=pl.BlockSpec((M, N), lambda i: (0, 0)),
        out_shape=jax.ShapeDtypeStruct((M, N), F32),
    )(x, w1, b1.reshape(1, H), w2, b2.reshape(1, H), w3p, b3p)


_NC, _NS = 2, 16
_NW = _NC * _NS


def _ring_gather(t_h, o_h, idx_all, base0, nch, bufs, sg, sw):
    nb = len(bufs)
    gd = [None] * nch
    wd = [None] * nch
    for j in range(nch):
        b = j % nb
        if j >= nb:
            wd[j - nb].wait()
        gd[j] = pltpu.async_copy(
            t_h.at[idx_all.at[j]], bufs[b], sg[b])
        if j >= 1:
            gd[j - 1].wait()
            wd[j - 1] = pltpu.async_copy(
                bufs[(j - 1) % nb],
                o_h.at[pl.ds(base0 + (j - 1) * 128, 128)], sw[(j - 1) % nb])
    gd[nch - 1].wait()
    wd[nch - 1] = pltpu.async_copy(
        bufs[(nch - 1) % nb],
        o_h.at[pl.ds(base0 + (nch - 1) * 128, 128)], sw[(nch - 1) % nb])
    for j in range(max(0, nch - nb), nch):
        wd[j].wait()


def _sc_gather2(tb, ib, tc, ic, nchunks):
    Ep = ib.shape[0]
    per_w = Ep // _NW
    nch = per_w // 128
    ib2, ic2 = ib.reshape(_NW, nch, 128), ic.reshape(_NW, nch, 128)
    mesh = plsc.VectorSubcoreMesh(core_axis_name="c", subcore_axis_name="s")

    @functools.partial(
        pl.kernel, mesh=mesh,
        out_type=(jax.ShapeDtypeStruct((Ep, H), F32),
                  jax.ShapeDtypeStruct((Ep, H), F32)),
        scratch_types=[pltpu.VMEM((nch, 128), jnp.int32),
                       pltpu.VMEM((128, H), F32),
                       pltpu.VMEM((128, H), F32),
                       pltpu.VMEM((128, H), F32),
                       pltpu.SemaphoreType.DMA,
                       pltpu.SemaphoreType.DMA,
                       pltpu.SemaphoreType.DMA,
                       pltpu.SemaphoreType.DMA,
                       pltpu.SemaphoreType.DMA,
                       pltpu.SemaphoreType.DMA],
    )
    def k(tb_h, ib_h, tc_h, ic_h, ob_h, oc_h, idx_all, r0, r1, r2,
          sg0, sg1, sg2, sw0, sw1, sw2):
        wid = lax.axis_index("s") * _NC + lax.axis_index("c")
        base0 = wid * per_w
        bufs, sg, sw = (r0, r1, r2), (sg0, sg1, sg2), (sw0, sw1, sw2)
        for t_h, i_h, o_h in ((tb_h, ib_h, ob_h), (tc_h, ic_h, oc_h)):
            pltpu.sync_copy(i_h.at[wid], idx_all)
            _ring_gather(t_h, o_h, idx_all, base0, nch, bufs, sg, sw)

    return k(tb, ib2, tc, ic2)


def _sc_bucket(dst, src, cap, rng):
    Ep = dst.shape[0]
    mesh = plsc.VectorSubcoreMesh(core_axis_name="c", subcore_axis_name="s")

    @functools.partial(
        pl.kernel, mesh=mesh,
        out_type=(jax.ShapeDtypeStruct((_NW * cap,), jnp.int32),
                  jax.ShapeDtypeStruct((_NW * cap,), jnp.int32),
                  jax.ShapeDtypeStruct((_NW * cap,), jnp.int32)),
        scratch_types=[pltpu.VMEM((Ep,), jnp.int32),
                       pltpu.VMEM((Ep,), jnp.int32),
                       pltpu.VMEM((cap + 16,), jnp.int32),
                       pltpu.VMEM((cap,), jnp.int32),
                       pltpu.VMEM((cap,), jnp.int32),
                       pltpu.SemaphoreType.DMA],
        compiler_params=pltpu.CompilerParams(needs_layout_passes=False),
    )
    def k(dst_h, src_h, p_h, dp_h, sp_h, dstv, srcv, pvm, dpv, spv, sem):
        w = lax.axis_index("s") * _NC + lax.axis_index("c")
        lo = w * rng
        pltpu.sync_copy(dst_h, dstv)
        pltpu.sync_copy(src_h, srcv)
        padv = jnp.full((16,), Ep - 1, jnp.int32)

        def initb(q, _):
            pvm[pl.ds(q * 16, 16)] = padv
            return 0

        lax.fori_loop(0, (cap + 16) // 16, initb, 0)
        lanes = lax.iota(jnp.int32, 16)

        def scan(i, off):
            d = dstv[pl.ds(i * 16, 16)]
            m = (d >= lo) & (d < lo + rng)
            cum = plsc.cumsum(m.astype(jnp.int32))
            pos = jnp.where(m, off + cum - 1, cap + 15)
            plsc.store_scatter(pvm, [pos], lanes + i * 16)
            return jnp.minimum(off + cum[15], cap)

        lax.fori_loop(0, Ep // 16, scan, 0)

        def permute(q, _):
            ids = pvm[pl.ds(q * 16, 16)]
            dpv[pl.ds(q * 16, 16)] = plsc.load_gather(dstv, [ids])
            spv[pl.ds(q * 16, 16)] = plsc.load_gather(srcv, [ids])
            return 0

        lax.fori_loop(0, cap // 16, permute, 0)
        pltpu.sync_copy(pvm.at[pl.ds(0, cap)], p_h.at[pl.ds(w * cap, cap)])
        pltpu.sync_copy(dpv, dp_h.at[pl.ds(w * cap, cap)])
        pltpu.sync_copy(spv, sp_h.at[pl.ds(w * cap, cap)])

    return k(dst, src)


def _sc_gather1(tb, ib, D, nchunks):
    Ep = ib.shape[0]
    per_w = Ep // _NW
    nch = per_w // 128
    ib2 = ib.reshape(_NW, nch, 128)
    mesh = plsc.VectorSubcoreMesh(core_axis_name="c", subcore_axis_name="s")

    @functools.partial(
        pl.kernel, mesh=mesh,
        out_type=jax.ShapeDtypeStruct((Ep, D), F32),
        scratch_types=[pltpu.VMEM((nch, 128), jnp.int32),
                       pltpu.VMEM((128, D), F32),
                       pltpu.VMEM((128, D), F32),
                       pltpu.VMEM((128, D), F32),
                       pltpu.SemaphoreType.DMA,
                       pltpu.SemaphoreType.DMA,
                       pltpu.SemaphoreType.DMA,
                       pltpu.SemaphoreType.DMA,
                       pltpu.SemaphoreType.DMA,
                       pltpu.SemaphoreType.DMA],
    )
    def k(t_h, i_h, o_h, idx_all, r0, r1, r2, sg0, sg1, sg2, sw0, sw1, sw2):
        wid = lax.axis_index("s") * _NC + lax.axis_index("c")
        base0 = wid * per_w
        pltpu.sync_copy(i_h.at[wid], idx_all)
        _ring_gather(t_h, o_h, idx_all, base0, nch,
                     (r0, r1, r2), (sg0, sg1, sg2), (sw0, sw1, sw2))

    return k(tb, ib2)


def _seg_body(rng, dp_ref, me_ref, o_ref):
    i = pl.program_id(0)
    cap = dp_ref.shape[2]
    rid = jax.lax.broadcasted_iota(jnp.int32, (rng, cap), 0) + i * rng
    oh = (rid == dp_ref[0]).astype(F32)
    o_ref[...] = _dot(oh, me_ref[...])


def _seg_tc(me, dp, n, rng):
    slots = me.shape[0]
    cap = slots // _NW
    dp3 = dp.reshape(_NW, 1, cap)
    return pl.pallas_call(
        functools.partial(_seg_body, rng),
        grid=(_NW,),
        in_specs=[pl.BlockSpec((1, 1, cap), lambda i: (i, 0, 0)),
                  pl.BlockSpec((cap, H), lambda i: (i, 0))],
        out_specs=pl.BlockSpec((rng, H), lambda i: (i, 0)),
        out_shape=jax.ShapeDtypeStruct((n, H), F32),
        compiler_params=pltpu.CompilerParams(dimension_semantics=("arbitrary",)),
    )(dp3, me)


def _padr(x, n, k=None):
    pc = 0 if k is None else k - x.shape[1]
    return jnp.pad(x, ((0, n - x.shape[0]), (0, pc)))


def _padi(idx, n, fill):
    return jnp.pad(idx, (0, n - idx.shape[0]), constant_values=fill).astype(jnp.int32)


def _split_edge_w(ps):
    (w1, b1), (w2, b2), (w3, b3) = ps
    return (w1[:H], w1[H:2 * H], w1[2 * H:], b1, w2, b2, w3, b3)


def _split_node_w(ps):
    (w1, b1), (w2, b2), (w3, b3) = ps
    return (w1[:H], w1[H:], b1, w2, b2, w3, b3)


def kernel(features, mesh_feats, g2m_attr, mm_attr, m2g_attr, params, g2m_src,
           g2m_dst, mm_src, mm_dst, m2g_src, m2g_dst):
    p = params
    NGp, NMp = 512, 5888
    RM, RG = NMp // _NW, NGp // _NW
    EGp, EMp, EDp = 1024, 36864, 1024
    CG, CM = 128, 1408

    feat = _padr(features[0], NGp, 80)

    gx = _embed3(feat, p['grid_embed'], bm=NGp)
    mx = _embed3(_padr(mesh_feats, NMp, 8), p['mesh_embed'], bm=736)

    eWa, eWb, eWc, eb1, eW2, eb2, eW3, eb3 = _split_edge_w(p['enc_edge'])
    dWa, dWb, dWc, db1, dW2, db2, dW3, db3 = _split_edge_w(p['dec_edge'])
    pe = [_split_edge_w(ps) for ps in p['proc_edge']]
    pn = [_split_node_w(ps) for ps in p['proc_node']]

    g2m_srcp = _padi(g2m_src, EGp, NGp - 1)
    g2m_dstp = _padi(g2m_dst, EGp, NMp - 1)
    mm_srcp = _padi(mm_src, EMp, NMp - 1)
    mm_dstp = _padi(mm_dst, EMp, NMp - 1)
    m2g_srcp = _padi(m2g_src, EDp, NMp - 1)
    m2g_dstp = _padi(m2g_dst, EDp, NGp - 1)

    Pg, DPg, SPg = _sc_bucket(g2m_dstp, g2m_srcp, CG, RM)
    Pm, DPm, SPm = _sc_bucket(mm_dstp, mm_srcp, CM, RM)
    Pd, DPd, SPd = _sc_bucket(m2g_dstp, m2g_srcp, CG, RG)

    ga16 = _sc_gather1(_padr(g2m_attr, EGp, 128), Pg, 128, 1)
    ma16 = _sc_gather1(_padr(mm_attr, EMp, 128), Pm, 128, CM // 128)
    da16 = _sc_gather1(_padr(m2g_attr, EDp, 128), Pd, 128, 1)
    ge = _embed3(ga16, p['g2m_edge_embed'], bm=512)
    me = _embed3(ma16, p['mm_edge_embed'], bm=512)
    de = _embed3(da16, p['m2g_edge_embed'], bm=512)

    gP = _proj(gx, jnp.concatenate([eWb, dWc], axis=1), bm=NGp)
    Pb_enc, Pc_dec = gP[:, :H], gP[:, H:]
    Pc_enc = _proj(mx, eWc, bm=736)

    gb, gc = _sc_gather2(Pb_enc, SPg, Pc_enc, DPg, nchunks=1)
    ge = _edge3(ge, gb, gc, eWa, eb1, eW2, eb2, eW3, eb3, bm=512)
    agg = _seg_tc(ge, DPg, NMp, RM)
    v1a, v1b, b1, v2, b2, v3, b3 = _split_node_w(p['enc_node'])
    mx, Pb, Pc = _node3(mx, agg, v1a, b1, v1b, v2, b2, v3, b3,
                        proj=(pe[0][1], pe[0][2]), bm=736)

    for i in range(9):
        wa, _, _, b1e, w2e, b2e, w3e, b3e = pe[i]
        gb, gc = _sc_gather2(Pb, SPm, Pc, DPm, nchunks=CM // 128)
        me = _edge3(me, gb, gc, wa, b1e, w2e, b2e, w3e, b3e, bm=512)
        agg = _seg_tc(me, DPm, NMp, RM)
        v1a, v1b, b1, v2, b2, v3, b3 = pn[i]
        nxt = (pe[i + 1][1], pe[i + 1][2]) if i < 8 else (dWb, dWb)
        mx, Pb, Pc = _node3(mx, agg, v1a, b1, v1b, v2, b2, v3, b3,
                            proj=nxt, bm=736)

    gb, gc = _sc_gather2(Pb, SPd, Pc_dec, DPd, nchunks=1)
    de = _edge3(de, gb, gc, dWa, db1, dW2, db2, dW3, db3, bm=512)
    agg = _seg_tc(de, DPd, NGp, RG)
    v1a, v1b, b1, v2, b2, v3, b3 = _split_node_w(p['dec_node'])
    gx = _node3(gx, agg, v1a, b1, v1b, v2, b2, v3, b3, proj=None, bm=NGp)

    out = _out3(gx, p['out'])
    return out[:288, :78][None]


# speedup vs baseline: 1.6077x; 1.6077x over previous
import functools

import jax
import jax.numpy as jnp
from jax import lax
from jax.experimental import pallas as pl
from jax.experimental.pallas import tpu as pltpu
from jax.experimental.pallas import tpu_sc as plsc

H = 256
F32 = jnp.float32


def _ln(h):
    m = jnp.mean(h, axis=-1, keepdims=True)
    c = h - m
    v = jnp.mean(c * c, axis=-1, keepdims=True)
    return c * lax.rsqrt(v + 1e-5)


def _dot(a, b):
    return jnp.dot(a, b, preferred_element_type=F32)


def _embed3_body(x_ref, w1, b1, w2, b2, w3, b3, o_ref):
    h = jnp.maximum(_dot(x_ref[...], w1[...]) + b1[...], 0.0)
    h = jnp.maximum(_dot(h, w2[...]) + b2[...], 0.0)
    h = _dot(h, w3[...]) + b3[...]
    o_ref[...] = _ln(h)


def _embed3(x, ps, bm):
    (w1, b1), (w2, b2), (w3, b3) = ps
    M, K = x.shape
    w1 = jnp.pad(w1, ((0, K - w1.shape[0]), (0, 0)))
    grid = M // bm
    wspec = lambda r, c: pl.BlockSpec((r, c), lambda i: (0, 0))
    return pl.pallas_call(
        _embed3_body,
        grid=(grid,),
        in_specs=[
            pl.BlockSpec((bm, K), lambda i: (i, 0)),
            wspec(K, H), wspec(1, H), wspec(H, H), wspec(1, H), wspec(H, H), wspec(1, H),
        ],
        out_specs=pl.BlockSpec((bm, H), lambda i: (i, 0)),
        out_shape=jax.ShapeDtypeStruct((M, H), F32),
        compiler_params=pltpu.CompilerParams(dimension_semantics=("arbitrary",)),
    )(x, w1, b1.reshape(1, H), w2, b2.reshape(1, H), w3, b3.reshape(1, H))


def _edge3_body(e_ref, gb_ref, gc_ref, w1a, b1, w2, b2, w3, b3, o_ref):
    e = e_ref[...]
    h = jnp.maximum(_dot(e, w1a[...]) + gb_ref[...] + gc_ref[...] + b1[...], 0.0)
    h = jnp.maximum(_dot(h, w2[...]) + b2[...], 0.0)
    h = _dot(h, w3[...]) + b3[...]
    o_ref[...] = e + _ln(h)


def _edge3(e, gb, gc, w1a, b1, w2, b2, w3, b3, bm):
    M = e.shape[0]
    grid = M // bm
    dspec = pl.BlockSpec((bm, H), lambda i: (i, 0))
    wspec = lambda r, c: pl.BlockSpec((r, c), lambda i: (0, 0))
    return pl.pallas_call(
        _edge3_body,
        grid=(grid,),
        in_specs=[dspec, dspec, dspec,
                  wspec(H, H), wspec(1, H), wspec(H, H), wspec(1, H), wspec(H, H), wspec(1, H)],
        out_specs=dspec,
        out_shape=jax.ShapeDtypeStruct((M, H), F32),
        compiler_params=pltpu.CompilerParams(dimension_semantics=("arbitrary",)),
    )(e, gb, gc, w1a, b1.reshape(1, H), w2, b2.reshape(1, H), w3, b3.reshape(1, H))


def _node3_body(nproj, x_ref, a_ref, v1a, v1b, b1, v2, b2, v3, b3, p1, p2, o_ref, pb_ref, pc_ref):
    x = x_ref[...]
    agg = a_ref[...]
    h = jnp.maximum(_dot(x, v1a[...]) + _dot(agg, v1b[...]) + b1[...], 0.0)
    h = jnp.maximum(_dot(h, v2[...]) + b2[...], 0.0)
    h = _dot(h, v3[...]) + b3[...]
    xn = x + _ln(h)
    o_ref[...] = xn
    if nproj:
        pb_ref[...] = _dot(xn, p1[...])
        pc_ref[...] = _dot(xn, p2[...])


def _node3_noproj_body(x_ref, a_ref, v1a, v1b, b1, v2, b2, v3, b3, o_ref):
    x = x_ref[...]
    agg = a_ref[...]
    h = jnp.maximum(_dot(x, v1a[...]) + _dot(agg, v1b[...]) + b1[...], 0.0)
    h = jnp.maximum(_dot(h, v2[...]) + b2[...], 0.0)
    h = _dot(h, v3[...]) + b3[...]
    o_ref[...] = x + _ln(h)


def _node3(x, a0, v1a, b1, v1b, v2, b2, v3, b3, proj, bm):
    M = x.shape[0]
    grid = M // bm
    dspec = pl.BlockSpec((bm, H), lambda i: (i, 0))
    wspec = lambda: pl.BlockSpec((H, H), lambda i: (0, 0))
    bspec = lambda: pl.BlockSpec((1, H), lambda i: (0, 0))
    if proj is None:
        return pl.pallas_call(
            _node3_noproj_body,
            grid=(grid,),
            in_specs=[dspec, dspec,
                      wspec(), wspec(), bspec(), wspec(), bspec(), wspec(), bspec()],
            out_specs=dspec,
            out_shape=jax.ShapeDtypeStruct((M, H), F32),
            compiler_params=pltpu.CompilerParams(dimension_semantics=("arbitrary",)),
        )(x, a0, v1a, v1b, b1.reshape(1, H), v2, b2.reshape(1, H), v3, b3.reshape(1, H))
    p1, p2 = proj
    return pl.pallas_call(
        functools.partial(_node3_body, True),
        grid=(grid,),
        in_specs=[dspec, dspec,
                  wspec(), wspec(), bspec(), wspec(), bspec(), wspec(), bspec(),
                  wspec(), wspec()],
        out_specs=(dspec, dspec, dspec),
        out_shape=(jax.ShapeDtypeStruct((M, H), F32),
                   jax.ShapeDtypeStruct((M, H), F32),
                   jax.ShapeDtypeStruct((M, H), F32)),
        compiler_params=pltpu.CompilerParams(dimension_semantics=("arbitrary",)),
    )(x, a0, v1a, v1b, b1.reshape(1, H), v2, b2.reshape(1, H), v3, b3.reshape(1, H), p1, p2)


def _proj_body(x_ref, w_ref, o_ref):
    o_ref[...] = _dot(x_ref[...], w_ref[...])


def _proj(x, w, bm):
    M = x.shape[0]
    N = w.shape[1]
    return pl.pallas_call(
        _proj_body,
        grid=(M // bm,),
        in_specs=[pl.BlockSpec((bm, H), lambda i: (i, 0)),
                  pl.BlockSpec((H, N), lambda i: (0, 0))],
        out_specs=pl.BlockSpec((bm, N), lambda i: (i, 0)),
        out_shape=jax.ShapeDtypeStruct((M, N), F32),
        compiler_params=pltpu.CompilerParams(dimension_semantics=("arbitrary",)),
    )(x, w)


def _out3_body(x_ref, w1, b1, w2, b2, w3, b3, o_ref):
    h = jnp.maximum(_dot(x_ref[...], w1[...]) + b1[...], 0.0)
    h = jnp.maximum(_dot(h, w2[...]) + b2[...], 0.0)
    o_ref[...] = _dot(h, w3[...]) + b3[...]


def _out3(x, ps):
    (w1, b1), (w2, b2), (w3, b3) = ps
    M = x.shape[0]
    N = 128
    w3p = jnp.pad(w3, ((0, 0), (0, N - w3.shape[1])))
    b3p = jnp.pad(b3, (0, N - b3.shape[0])).reshape(1, N)
    wspec = lambda r, c: pl.BlockSpec((r, c), lambda i: (0, 0))
    return pl.pallas_call(
        _out3_body,
        grid=(1,),
        in_specs=[pl.BlockSpec((M, H), lambda i: (0, 0)),
                  wspec(H, H), wspec(1, H), wspec(H, H), wspec(1, H), wspec(H, N), wspec(1, N)],
        out_specs=pl.BlockSpec((M, N), lambda i: (0, 0)),
        out_shape=jax.ShapeDtypeStruct((M, N), F32),
    )(x, w1, b1.reshape(1, H), w2, b2.reshape(1, H), w3p, b3p)


_NC, _NS = 2, 16
_NW = _NC * _NS


def _ring_gather(t_h, o_h, idx_all, base0, nch, bufs, sg, sw):
    nb = len(bufs)
    gd = [None] * nch
    wd = [None] * nch
    for j in range(nch):
        b = j % nb
        if j >= nb:
            wd[j - nb].wait()
        gd[j] = pltpu.async_copy(
            t_h.at[idx_all.at[j]], bufs[b], sg[b])
        if j >= 1:
            gd[j - 1].wait()
            wd[j - 1] = pltpu.async_copy(
                bufs[(j - 1) % nb],
                o_h.at[pl.ds(base0 + (j - 1) * 128, 128)], sw[(j - 1) % nb])
    gd[nch - 1].wait()
    wd[nch - 1] = pltpu.async_copy(
        bufs[(nch - 1) % nb],
        o_h.at[pl.ds(base0 + (nch - 1) * 128, 128)], sw[(nch - 1) % nb])
    for j in range(max(0, nch - nb), nch):
        wd[j].wait()


def _sc_gather2(tb, ib, tc, ic, nchunks):
    Ep = ib.shape[0]
    per_w = Ep // _NW
    nch = per_w // 128
    ib2, ic2 = ib.reshape(_NW, nch, 128), ic.reshape(_NW, nch, 128)
    mesh = plsc.VectorSubcoreMesh(core_axis_name="c", subcore_axis_name="s")

    @functools.partial(
        pl.kernel, mesh=mesh,
        out_type=(jax.ShapeDtypeStruct((Ep, H), F32),
                  jax.ShapeDtypeStruct((Ep, H), F32)),
        scratch_types=[pltpu.VMEM((nch, 128), jnp.int32),
                       pltpu.VMEM((128, H), F32),
                       pltpu.VMEM((128, H), F32),
                       pltpu.VMEM((128, H), F32),
                       pltpu.SemaphoreType.DMA,
                       pltpu.SemaphoreType.DMA,
                       pltpu.SemaphoreType.DMA,
                       pltpu.SemaphoreType.DMA,
                       pltpu.SemaphoreType.DMA,
                       pltpu.SemaphoreType.DMA],
    )
    def k(tb_h, ib_h, tc_h, ic_h, ob_h, oc_h, idx_all, r0, r1, r2,
          sg0, sg1, sg2, sw0, sw1, sw2):
        wid = lax.axis_index("s") * _NC + lax.axis_index("c")
        base0 = wid * per_w
        bufs, sg, sw = (r0, r1, r2), (sg0, sg1, sg2), (sw0, sw1, sw2)
        for t_h, i_h, o_h in ((tb_h, ib_h, ob_h), (tc_h, ic_h, oc_h)):
            pltpu.sync_copy(i_h.at[wid], idx_all)
            _ring_gather(t_h, o_h, idx_all, base0, nch, bufs, sg, sw)

    return k(tb, ib2, tc, ic2)


def _sc_bucket(dst, src, cap, rng):
    Ep = dst.shape[0]
    mesh = plsc.VectorSubcoreMesh(core_axis_name="c", subcore_axis_name="s")

    @functools.partial(
        pl.kernel, mesh=mesh,
        out_type=(jax.ShapeDtypeStruct((_NW * cap,), jnp.int32),
                  jax.ShapeDtypeStruct((_NW * cap,), jnp.int32),
                  jax.ShapeDtypeStruct((_NW * cap,), jnp.int32)),
        scratch_types=[pltpu.VMEM((Ep,), jnp.int32),
                       pltpu.VMEM((Ep,), jnp.int32),
                       pltpu.VMEM((cap + 16,), jnp.int32),
                       pltpu.VMEM((cap,), jnp.int32),
                       pltpu.VMEM((cap,), jnp.int32),
                       pltpu.SemaphoreType.DMA],
        compiler_params=pltpu.CompilerParams(needs_layout_passes=False),
    )
    def k(dst_h, src_h, p_h, dp_h, sp_h, dstv, srcv, pvm, dpv, spv, sem):
        w = lax.axis_index("s") * _NC + lax.axis_index("c")
        lo = w * rng
        pltpu.sync_copy(dst_h, dstv)
        pltpu.sync_copy(src_h, srcv)
        padv = jnp.full((16,), Ep - 1, jnp.int32)

        def initb(q, _):
            pvm[pl.ds(q * 16, 16)] = padv
            return 0

        lax.fori_loop(0, (cap + 16) // 16, initb, 0)
        lanes = lax.iota(jnp.int32, 16)

        def scan(i, off):
            d = dstv[pl.ds(i * 16, 16)]
            m = (d >= lo) & (d < lo + rng)
            cum = plsc.cumsum(m.astype(jnp.int32))
            pos = jnp.where(m, off + cum - 1, cap + 15)
            plsc.store_scatter(pvm, [pos], lanes + i * 16)
            return jnp.minimum(off + cum[15], cap)

        lax.fori_loop(0, Ep // 16, scan, 0)

        def permute(q, _):
            ids = pvm[pl.ds(q * 16, 16)]
            dpv[pl.ds(q * 16, 16)] = plsc.load_gather(dstv, [ids])
            spv[pl.ds(q * 16, 16)] = plsc.load_gather(srcv, [ids])
            return 0

        lax.fori_loop(0, cap // 16, permute, 0)
        pltpu.sync_copy(pvm.at[pl.ds(0, cap)], p_h.at[pl.ds(w * cap, cap)])
        pltpu.sync_copy(dpv, dp_h.at[pl.ds(w * cap, cap)])
        pltpu.sync_copy(spv, sp_h.at[pl.ds(w * cap, cap)])

    return k(dst, src)


def _sc_gather1(tb, ib, D, nchunks):
    Ep = ib.shape[0]
    per_w = Ep // _NW
    nch = per_w // 128
    ib2 = ib.reshape(_NW, nch, 128)
    mesh = plsc.VectorSubcoreMesh(core_axis_name="c", subcore_axis_name="s")

    @functools.partial(
        pl.kernel, mesh=mesh,
        out_type=jax.ShapeDtypeStruct((Ep, D), F32),
        scratch_types=[pltpu.VMEM((nch, 128), jnp.int32),
                       pltpu.VMEM((128, D), F32),
                       pltpu.VMEM((128, D), F32),
                       pltpu.VMEM((128, D), F32),
                       pltpu.SemaphoreType.DMA,
                       pltpu.SemaphoreType.DMA,
                       pltpu.SemaphoreType.DMA,
                       pltpu.SemaphoreType.DMA,
                       pltpu.SemaphoreType.DMA,
                       pltpu.SemaphoreType.DMA],
    )
    def k(t_h, i_h, o_h, idx_all, r0, r1, r2, sg0, sg1, sg2, sw0, sw1, sw2):
        wid = lax.axis_index("s") * _NC + lax.axis_index("c")
        base0 = wid * per_w
        pltpu.sync_copy(i_h.at[wid], idx_all)
        _ring_gather(t_h, o_h, idx_all, base0, nch,
                     (r0, r1, r2), (sg0, sg1, sg2), (sw0, sw1, sw2))

    return k(tb, ib2)


def _seg_body(rng, dp_ref, me_ref, o_ref):
    i = pl.program_id(0)
    cap = dp_ref.shape[2]
    rid = jax.lax.broadcasted_iota(jnp.int32, (rng, cap), 0) + i * rng
    oh = (rid == dp_ref[0]).astype(F32)
    o_ref[...] = _dot(oh, me_ref[...])


def _seg_tc(me, dp, n, rng):
    slots = me.shape[0]
    cap = slots // _NW
    dp3 = dp.reshape(_NW, 1, cap)
    return pl.pallas_call(
        functools.partial(_seg_body, rng),
        grid=(_NW,),
        in_specs=[pl.BlockSpec((1, 1, cap), lambda i: (i, 0, 0)),
                  pl.BlockSpec((cap, H), lambda i: (i, 0))],
        out_specs=pl.BlockSpec((rng, H), lambda i: (i, 0)),
        out_shape=jax.ShapeDtypeStruct((n, H), F32),
        compiler_params=pltpu.CompilerParams(dimension_semantics=("arbitrary",)),
    )(dp3, me)


def _gtc_body(rng, dp_ref, pc_ref, o_ref):
    i = pl.program_id(0)
    cap = dp_ref.shape[2]
    rid = jax.lax.broadcasted_iota(jnp.int32, (cap, rng), 1) + i * rng
    oh = (jnp.reshape(dp_ref[0], (cap, 1)) == rid).astype(F32)
    o_ref[...] = _dot(oh, pc_ref[...])


def _gather_dst_tc(pc, dp, rng):
    slots = dp.shape[0]
    cap = slots // _NW
    dp3 = dp.reshape(_NW, 1, cap)
    return pl.pallas_call(
        functools.partial(_gtc_body, rng),
        grid=(_NW,),
        in_specs=[pl.BlockSpec((1, 1, cap), lambda i: (i, 0, 0)),
                  pl.BlockSpec((rng, H), lambda i: (i, 0))],
        out_specs=pl.BlockSpec((cap, H), lambda i: (i, 0)),
        out_shape=jax.ShapeDtypeStruct((slots, H), F32),
        compiler_params=pltpu.CompilerParams(dimension_semantics=("arbitrary",)),
    )(dp3, pc)


def _padr(x, n, k=None):
    pc = 0 if k is None else k - x.shape[1]
    return jnp.pad(x, ((0, n - x.shape[0]), (0, pc)))


def _padi(idx, n, fill):
    return jnp.pad(idx, (0, n - idx.shape[0]), constant_values=fill).astype(jnp.int32)


def _split_edge_w(ps):
    (w1, b1), (w2, b2), (w3, b3) = ps
    return (w1[:H], w1[H:2 * H], w1[2 * H:], b1, w2, b2, w3, b3)


def _split_node_w(ps):
    (w1, b1), (w2, b2), (w3, b3) = ps
    return (w1[:H], w1[H:], b1, w2, b2, w3, b3)


def kernel(features, mesh_feats, g2m_attr, mm_attr, m2g_attr, params, g2m_src,
           g2m_dst, mm_src, mm_dst, m2g_src, m2g_dst):
    p = params
    NGp, NMp = 512, 5888
    RM, RG = NMp // _NW, NGp // _NW
    EGp, EMp, EDp = 1024, 36864, 1024
    CG, CM = 128, 1408

    feat = _padr(features[0], NGp, 80)

    gx = _embed3(feat, p['grid_embed'], bm=NGp)
    mx = _embed3(_padr(mesh_feats, NMp, 8), p['mesh_embed'], bm=736)

    eWa, eWb, eWc, eb1, eW2, eb2, eW3, eb3 = _split_edge_w(p['enc_edge'])
    dWa, dWb, dWc, db1, dW2, db2, dW3, db3 = _split_edge_w(p['dec_edge'])
    pe = [_split_edge_w(ps) for ps in p['proc_edge']]
    pn = [_split_node_w(ps) for ps in p['proc_node']]

    g2m_srcp = _padi(g2m_src, EGp, NGp - 1)
    g2m_dstp = _padi(g2m_dst, EGp, NMp - 1)
    mm_srcp = _padi(mm_src, EMp, NMp - 1)
    mm_dstp = _padi(mm_dst, EMp, NMp - 1)
    m2g_srcp = _padi(m2g_src, EDp, NMp - 1)
    m2g_dstp = _padi(m2g_dst, EDp, NGp - 1)

    Pg, DPg, SPg = _sc_bucket(g2m_dstp, g2m_srcp, CG, RM)
    Pm, DPm, SPm = _sc_bucket(mm_dstp, mm_srcp, CM, RM)
    Pd, DPd, SPd = _sc_bucket(m2g_dstp, m2g_srcp, CG, RG)

    ga16 = _sc_gather1(_padr(g2m_attr, EGp, 128), Pg, 128, 1)
    ma16 = _sc_gather1(_padr(mm_attr, EMp, 128), Pm, 128, CM // 128)
    da16 = _sc_gather1(_padr(m2g_attr, EDp, 128), Pd, 128, 1)
    ge = _embed3(ga16, p['g2m_edge_embed'], bm=512)
    me = _embed3(ma16, p['mm_edge_embed'], bm=512)
    de = _embed3(da16, p['m2g_edge_embed'], bm=512)

    gP = _proj(gx, jnp.concatenate([eWb, dWc], axis=1), bm=NGp)
    Pb_enc, Pc_dec = gP[:, :H], gP[:, H:]
    Pc_enc = _proj(mx, eWc, bm=736)

    gb = _sc_gather1(Pb_enc, SPg, H, 1)
    gc = _gather_dst_tc(Pc_enc, DPg, RM)
    ge = _edge3(ge, gb, gc, eWa, eb1, eW2, eb2, eW3, eb3, bm=512)
    agg = _seg_tc(ge, DPg, NMp, RM)
    v1a, v1b, b1, v2, b2, v3, b3 = _split_node_w(p['enc_node'])
    mx, Pb, Pc = _node3(mx, agg, v1a, b1, v1b, v2, b2, v3, b3,
                        proj=(pe[0][1], pe[0][2]), bm=736)

    for i in range(9):
        wa, _, _, b1e, w2e, b2e, w3e, b3e = pe[i]
        gb = _sc_gather1(Pb, SPm, H, CM // 128)
        gc = _gather_dst_tc(Pc, DPm, RM)
        me = _edge3(me, gb, gc, wa, b1e, w2e, b2e, w3e, b3e, bm=512)
        agg = _seg_tc(me, DPm, NMp, RM)
        v1a, v1b, b1, v2, b2, v3, b3 = pn[i]
        nxt = (pe[i + 1][1], pe[i + 1][2]) if i < 8 else (dWb, dWb)
        mx, Pb, Pc = _node3(mx, agg, v1a, b1, v1b, v2, b2, v3, b3,
                            proj=nxt, bm=736)

    gb = _sc_gather1(Pb, SPd, H, 1)
    gc = _gather_dst_tc(Pc_dec, DPd, RG)
    de = _edge3(de, gb, gc, dWa, db1, dW2, db2, dW3, db3, bm=512)
    agg = _seg_tc(de, DPd, NGp, RG)
    v1a, v1b, b1, v2, b2, v3, b3 = _split_node_w(p['dec_node'])
    gx = _node3(gx, agg, v1a, b1, v1b, v2, b2, v3, b3, proj=None, bm=NGp)

    out = _out3(gx, p['out'])
    return out[:288, :78][None]


# speedup vs baseline: 4.4461x; 2.7655x over previous
import functools

import jax
import jax.numpy as jnp
from jax import lax
from jax.experimental import pallas as pl
from jax.experimental.pallas import tpu as pltpu
from jax.experimental.pallas import tpu_sc as plsc

H = 256
F32 = jnp.float32


def _ln(h):
    m = jnp.mean(h, axis=-1, keepdims=True)
    c = h - m
    v = jnp.mean(c * c, axis=-1, keepdims=True)
    return c * lax.rsqrt(v + 1e-5)


def _dot(a, b):
    return jnp.dot(a, b, preferred_element_type=F32)


def _embed3_body(x_ref, w1, b1, w2, b2, w3, b3, o_ref):
    h = jnp.maximum(_dot(x_ref[...], w1[...]) + b1[...], 0.0)
    h = jnp.maximum(_dot(h, w2[...]) + b2[...], 0.0)
    h = _dot(h, w3[...]) + b3[...]
    o_ref[...] = _ln(h)


def _embed3(x, ps, bm):
    (w1, b1), (w2, b2), (w3, b3) = ps
    M, K = x.shape
    w1 = jnp.pad(w1, ((0, K - w1.shape[0]), (0, 0)))
    grid = M // bm
    wspec = lambda r, c: pl.BlockSpec((r, c), lambda i: (0, 0))
    return pl.pallas_call(
        _embed3_body,
        grid=(grid,),
        in_specs=[
            pl.BlockSpec((bm, K), lambda i: (i, 0)),
            wspec(K, H), wspec(1, H), wspec(H, H), wspec(1, H), wspec(H, H), wspec(1, H),
        ],
        out_specs=pl.BlockSpec((bm, H), lambda i: (i, 0)),
        out_shape=jax.ShapeDtypeStruct((M, H), F32),
        compiler_params=pltpu.CompilerParams(dimension_semantics=("arbitrary",)),
    )(x, w1, b1.reshape(1, H), w2, b2.reshape(1, H), w3, b3.reshape(1, H))


def _edge3_body(e_ref, gb_ref, gc_ref, w1a, b1, w2, b2, w3, b3, o_ref):
    e = e_ref[...]
    h = jnp.maximum(_dot(e, w1a[...]) + gb_ref[...] + gc_ref[...] + b1[...], 0.0)
    h = jnp.maximum(_dot(h, w2[...]) + b2[...], 0.0)
    h = _dot(h, w3[...]) + b3[...]
    o_ref[...] = e + _ln(h)


def _edge3(e, gb, gc, w1a, b1, w2, b2, w3, b3, bm):
    M = e.shape[0]
    grid = M // bm
    dspec = pl.BlockSpec((bm, H), lambda i: (i, 0))
    wspec = lambda r, c: pl.BlockSpec((r, c), lambda i: (0, 0))
    return pl.pallas_call(
        _edge3_body,
        grid=(grid,),
        in_specs=[dspec, dspec, dspec,
                  wspec(H, H), wspec(1, H), wspec(H, H), wspec(1, H), wspec(H, H), wspec(1, H)],
        out_specs=dspec,
        out_shape=jax.ShapeDtypeStruct((M, H), F32),
        compiler_params=pltpu.CompilerParams(dimension_semantics=("arbitrary",)),
    )(e, gb, gc, w1a, b1.reshape(1, H), w2, b2.reshape(1, H), w3, b3.reshape(1, H))


def _node3_body(nproj, x_ref, a_ref, v1a, v1b, b1, v2, b2, v3, b3, p1, p2, o_ref, pb_ref, pc_ref):
    x = x_ref[...]
    agg = a_ref[...]
    h = jnp.maximum(_dot(x, v1a[...]) + _dot(agg, v1b[...]) + b1[...], 0.0)
    h = jnp.maximum(_dot(h, v2[...]) + b2[...], 0.0)
    h = _dot(h, v3[...]) + b3[...]
    xn = x + _ln(h)
    o_ref[...] = xn
    if nproj:
        pb_ref[...] = _dot(xn, p1[...])
        pc_ref[...] = _dot(xn, p2[...])


def _node3_noproj_body(x_ref, a_ref, v1a, v1b, b1, v2, b2, v3, b3, o_ref):
    x = x_ref[...]
    agg = a_ref[...]
    h = jnp.maximum(_dot(x, v1a[...]) + _dot(agg, v1b[...]) + b1[...], 0.0)
    h = jnp.maximum(_dot(h, v2[...]) + b2[...], 0.0)
    h = _dot(h, v3[...]) + b3[...]
    o_ref[...] = x + _ln(h)


def _node3(x, a0, v1a, b1, v1b, v2, b2, v3, b3, proj, bm):
    M = x.shape[0]
    grid = M // bm
    dspec = pl.BlockSpec((bm, H), lambda i: (i, 0))
    wspec = lambda: pl.BlockSpec((H, H), lambda i: (0, 0))
    bspec = lambda: pl.BlockSpec((1, H), lambda i: (0, 0))
    if proj is None:
        return pl.pallas_call(
            _node3_noproj_body,
            grid=(grid,),
            in_specs=[dspec, dspec,
                      wspec(), wspec(), bspec(), wspec(), bspec(), wspec(), bspec()],
            out_specs=dspec,
            out_shape=jax.ShapeDtypeStruct((M, H), F32),
            compiler_params=pltpu.CompilerParams(dimension_semantics=("arbitrary",)),
        )(x, a0, v1a, v1b, b1.reshape(1, H), v2, b2.reshape(1, H), v3, b3.reshape(1, H))
    p1, p2 = proj
    return pl.pallas_call(
        functools.partial(_node3_body, True),
        grid=(grid,),
        in_specs=[dspec, dspec,
                  wspec(), wspec(), bspec(), wspec(), bspec(), wspec(), bspec(),
                  wspec(), wspec()],
        out_specs=(dspec, dspec, dspec),
        out_shape=(jax.ShapeDtypeStruct((M, H), F32),
                   jax.ShapeDtypeStruct((M, H), F32),
                   jax.ShapeDtypeStruct((M, H), F32)),
        compiler_params=pltpu.CompilerParams(dimension_semantics=("arbitrary",)),
    )(x, a0, v1a, v1b, b1.reshape(1, H), v2, b2.reshape(1, H), v3, b3.reshape(1, H), p1, p2)


def _proj_body(x_ref, w_ref, o_ref):
    o_ref[...] = _dot(x_ref[...], w_ref[...])


def _proj(x, w, bm):
    M = x.shape[0]
    N = w.shape[1]
    return pl.pallas_call(
        _proj_body,
        grid=(M // bm,),
        in_specs=[pl.BlockSpec((bm, H), lambda i: (i, 0)),
                  pl.BlockSpec((H, N), lambda i: (0, 0))],
        out_specs=pl.BlockSpec((bm, N), lambda i: (i, 0)),
        out_shape=jax.ShapeDtypeStruct((M, N), F32),
        compiler_params=pltpu.CompilerParams(dimension_semantics=("arbitrary",)),
    )(x, w)


def _out3_body(x_ref, w1, b1, w2, b2, w3, b3, o_ref):
    h = jnp.maximum(_dot(x_ref[...], w1[...]) + b1[...], 0.0)
    h = jnp.maximum(_dot(h, w2[...]) + b2[...], 0.0)
    o_ref[...] = _dot(h, w3[...]) + b3[...]


def _out3(x, ps):
    (w1, b1), (w2, b2), (w3, b3) = ps
    M = x.shape[0]
    N = 128
    w3p = jnp.pad(w3, ((0, 0), (0, N - w3.shape[1])))
    b3p = jnp.pad(b3, (0, N - b3.shape[0])).reshape(1, N)
    wspec = lambda r, c: pl.BlockSpec((r, c), lambda i: (0, 0))
    return pl.pallas_call(
        _out3_body,
        grid=(1,),
        in_specs=[pl.BlockSpec((M, H), lambda i: (0, 0)),
                  wspec(H, H), wspec(1, H), wspec(H, H), wspec(1, H), wspec(H, N), wspec(1, N)],
        out_specs=pl.BlockSpec((M, N), lambda i: (0, 0)),
        out_shape=jax.ShapeDtypeStruct((M, N), F32),
    )(x, w1, b1.reshape(1, H), w2, b2.reshape(1, H), w3p, b3p)


_NC, _NS = 2, 16
_NW = _NC * _NS


def _ring_gather(t_h, o_h, idx_all, base0, nch, bufs, sg, sw):
    nb = len(bufs)
    gd = [None] * nch
    wd = [None] * nch
    for j in range(nch):
        b = j % nb
        if j >= nb:
            wd[j - nb].wait()
        gd[j] = pltpu.async_copy(
            t_h.at[idx_all.at[j]], bufs[b], sg[b])
        if j >= 1:
            gd[j - 1].wait()
            wd[j - 1] = pltpu.async_copy(
                bufs[(j - 1) % nb],
                o_h.at[pl.ds(base0 + (j - 1) * 128, 128)], sw[(j - 1) % nb])
    gd[nch - 1].wait()
    wd[nch - 1] = pltpu.async_copy(
        bufs[(nch - 1) % nb],
        o_h.at[pl.ds(base0 + (nch - 1) * 128, 128)], sw[(nch - 1) % nb])
    for j in range(max(0, nch - nb), nch):
        wd[j].wait()


def _sc_gather2(tb, ib, tc, ic, nchunks):
    Ep = ib.shape[0]
    per_w = Ep // _NW
    nch = per_w // 128
    ib2, ic2 = ib.reshape(_NW, nch, 128), ic.reshape(_NW, nch, 128)
    mesh = plsc.VectorSubcoreMesh(core_axis_name="c", subcore_axis_name="s")

    @functools.partial(
        pl.kernel, mesh=mesh,
        out_type=(jax.ShapeDtypeStruct((Ep, H), F32),
                  jax.ShapeDtypeStruct((Ep, H), F32)),
        scratch_types=[pltpu.VMEM((nch, 128), jnp.int32),
                       pltpu.VMEM((128, H), F32),
                       pltpu.VMEM((128, H), F32),
                       pltpu.VMEM((128, H), F32),
                       pltpu.SemaphoreType.DMA,
                       pltpu.SemaphoreType.DMA,
                       pltpu.SemaphoreType.DMA,
                       pltpu.SemaphoreType.DMA,
                       pltpu.SemaphoreType.DMA,
                       pltpu.SemaphoreType.DMA],
    )
    def k(tb_h, ib_h, tc_h, ic_h, ob_h, oc_h, idx_all, r0, r1, r2,
          sg0, sg1, sg2, sw0, sw1, sw2):
        wid = lax.axis_index("s") * _NC + lax.axis_index("c")
        base0 = wid * per_w
        bufs, sg, sw = (r0, r1, r2), (sg0, sg1, sg2), (sw0, sw1, sw2)
        for t_h, i_h, o_h in ((tb_h, ib_h, ob_h), (tc_h, ic_h, oc_h)):
            pltpu.sync_copy(i_h.at[wid], idx_all)
            _ring_gather(t_h, o_h, idx_all, base0, nch, bufs, sg, sw)

    return k(tb, ib2, tc, ic2)


def _sc_bucket(dst, src, cap, rng, e_real):
    Ep = dst.shape[0]
    mesh = plsc.VectorSubcoreMesh(core_axis_name="c", subcore_axis_name="s")

    @functools.partial(
        pl.kernel, mesh=mesh,
        out_type=(jax.ShapeDtypeStruct((_NW * cap,), jnp.int32),
                  jax.ShapeDtypeStruct((_NW * cap,), jnp.int32),
                  jax.ShapeDtypeStruct((_NW * cap,), jnp.int32)),
        scratch_types=[pltpu.VMEM((Ep,), jnp.int32),
                       pltpu.VMEM((Ep,), jnp.int32),
                       pltpu.VMEM((cap + 16,), jnp.int32),
                       pltpu.VMEM((cap,), jnp.int32),
                       pltpu.VMEM((cap,), jnp.int32),
                       pltpu.SemaphoreType.DMA],
        compiler_params=pltpu.CompilerParams(needs_layout_passes=False),
    )
    def k(dst_h, src_h, p_h, dp_h, sp_h, dstv, srcv, pvm, dpv, spv, sem):
        w = lax.axis_index("s") * _NC + lax.axis_index("c")
        lo = w * rng
        pltpu.sync_copy(dst_h, dstv)
        pltpu.sync_copy(src_h, srcv)
        npad = Ep - e_real
        lanes0 = lax.iota(jnp.int32, 16)

        def initb(q, _):
            pvm[pl.ds(q * 16, 16)] = e_real + (q * 16 + lanes0) % npad
            return 0

        lax.fori_loop(0, (cap + 16) // 16, initb, 0)
        lanes = lax.iota(jnp.int32, 16)

        def scan(i, off):
            d = dstv[pl.ds(i * 16, 16)]
            m = (d >= lo) & (d < lo + rng)
            cum = plsc.cumsum(m.astype(jnp.int32))
            pos = jnp.where(m, off + cum - 1, cap + 15)
            plsc.store_scatter(pvm, [pos], lanes + i * 16)
            return jnp.minimum(off + cum[15], cap)

        lax.fori_loop(0, Ep // 16, scan, 0)

        def permute(q, _):
            ids = pvm[pl.ds(q * 16, 16)]
            dpv[pl.ds(q * 16, 16)] = plsc.load_gather(dstv, [ids])
            spv[pl.ds(q * 16, 16)] = plsc.load_gather(srcv, [ids])
            return 0

        lax.fori_loop(0, cap // 16, permute, 0)
        pltpu.sync_copy(pvm.at[pl.ds(0, cap)], p_h.at[pl.ds(w * cap, cap)])
        pltpu.sync_copy(dpv, dp_h.at[pl.ds(w * cap, cap)])
        pltpu.sync_copy(spv, sp_h.at[pl.ds(w * cap, cap)])

    return k(dst, src)


def _sc_gather1(tb, ib, D, nchunks):
    Ep = ib.shape[0]
    per_w = Ep // _NW
    nch = per_w // 128
    ib2 = ib.reshape(_NW, nch, 128)
    mesh = plsc.VectorSubcoreMesh(core_axis_name="c", subcore_axis_name="s")

    @functools.partial(
        pl.kernel, mesh=mesh,
        out_type=jax.ShapeDtypeStruct((Ep, D), F32),
        scratch_types=[pltpu.VMEM((nch, 128), jnp.int32),
                       pltpu.VMEM((128, D), F32),
                       pltpu.VMEM((128, D), F32),
                       pltpu.VMEM((128, D), F32),
                       pltpu.SemaphoreType.DMA,
                       pltpu.SemaphoreType.DMA,
                       pltpu.SemaphoreType.DMA,
                       pltpu.SemaphoreType.DMA,
                       pltpu.SemaphoreType.DMA,
                       pltpu.SemaphoreType.DMA],
    )
    def k(t_h, i_h, o_h, idx_all, r0, r1, r2, sg0, sg1, sg2, sw0, sw1, sw2):
        wid = lax.axis_index("s") * _NC + lax.axis_index("c")
        base0 = wid * per_w
        pltpu.sync_copy(i_h.at[wid], idx_all)
        _ring_gather(t_h, o_h, idx_all, base0, nch,
                     (r0, r1, r2), (sg0, sg1, sg2), (sw0, sw1, sw2))

    return k(tb, ib2)


def _seg_body(rng, dp_ref, me_ref, o_ref):
    i = pl.program_id(0)
    cap = dp_ref.shape[2]
    rid = jax.lax.broadcasted_iota(jnp.int32, (rng, cap), 0) + i * rng
    oh = (rid == dp_ref[0]).astype(F32)
    o_ref[...] = _dot(oh, me_ref[...])


def _seg_tc(me, dp, n, rng):
    slots = me.shape[0]
    cap = slots // _NW
    dp3 = dp.reshape(_NW, 1, cap)
    return pl.pallas_call(
        functools.partial(_seg_body, rng),
        grid=(_NW,),
        in_specs=[pl.BlockSpec((1, 1, cap), lambda i: (i, 0, 0)),
                  pl.BlockSpec((cap, H), lambda i: (i, 0))],
        out_specs=pl.BlockSpec((rng, H), lambda i: (i, 0)),
        out_shape=jax.ShapeDtypeStruct((n, H), F32),
        compiler_params=pltpu.CompilerParams(dimension_semantics=("arbitrary",)),
    )(dp3, me)


def _gtc_body(rng, dp_ref, pc_ref, o_ref):
    i = pl.program_id(0)
    cap = dp_ref.shape[2]
    rid = jax.lax.broadcasted_iota(jnp.int32, (cap, rng), 1) + i * rng
    oh = (jnp.reshape(dp_ref[0], (cap, 1)) == rid).astype(F32)
    o_ref[...] = _dot(oh, pc_ref[...])


def _gather_dst_tc(pc, dp, rng):
    slots = dp.shape[0]
    cap = slots // _NW
    dp3 = dp.reshape(_NW, 1, cap)
    return pl.pallas_call(
        functools.partial(_gtc_body, rng),
        grid=(_NW,),
        in_specs=[pl.BlockSpec((1, 1, cap), lambda i: (i, 0, 0)),
                  pl.BlockSpec((rng, H), lambda i: (i, 0))],
        out_specs=pl.BlockSpec((cap, H), lambda i: (i, 0)),
        out_shape=jax.ShapeDtypeStruct((slots, H), F32),
        compiler_params=pltpu.CompilerParams(dimension_semantics=("arbitrary",)),
    )(dp3, pc)


def _padr(x, n, k=None):
    pc = 0 if k is None else k - x.shape[1]
    return jnp.pad(x, ((0, n - x.shape[0]), (0, pc)))


def _padi(idx, n, fill):
    return jnp.pad(idx, (0, n - idx.shape[0]), constant_values=fill).astype(jnp.int32)


def _padi_spread(idx, n, mod):
    e = idx.shape[0]
    sp = (jnp.arange(n - e, dtype=jnp.int32) * 97 + 13) % mod
    return jnp.concatenate([idx.astype(jnp.int32), sp])


def _split_edge_w(ps):
    (w1, b1), (w2, b2), (w3, b3) = ps
    return (w1[:H], w1[H:2 * H], w1[2 * H:], b1, w2, b2, w3, b3)


def _split_node_w(ps):
    (w1, b1), (w2, b2), (w3, b3) = ps
    return (w1[:H], w1[H:], b1, w2, b2, w3, b3)


def kernel(features, mesh_feats, g2m_attr, mm_attr, m2g_attr, params, g2m_src,
           g2m_dst, mm_src, mm_dst, m2g_src, m2g_dst):
    p = params
    NGp, NMp = 512, 5888
    RM, RG = NMp // _NW, NGp // _NW
    EGp, EMp, EDp = 1024, 36864, 1024
    CG, CM = 128, 1408

    feat = _padr(features[0], NGp, 80)

    gx = _embed3(feat, p['grid_embed'], bm=NGp)
    mx = _embed3(_padr(mesh_feats, NMp, 8), p['mesh_embed'], bm=736)

    eWa, eWb, eWc, eb1, eW2, eb2, eW3, eb3 = _split_edge_w(p['enc_edge'])
    dWa, dWb, dWc, db1, dW2, db2, dW3, db3 = _split_edge_w(p['dec_edge'])
    pe = [_split_edge_w(ps) for ps in p['proc_edge']]
    pn = [_split_node_w(ps) for ps in p['proc_node']]

    g2m_srcp = _padi_spread(g2m_src, EGp, NGp)
    g2m_dstp = _padi(g2m_dst, EGp, NMp - 1)
    mm_srcp = _padi_spread(mm_src, EMp, NMp)
    mm_dstp = _padi(mm_dst, EMp, NMp - 1)
    m2g_srcp = _padi_spread(m2g_src, EDp, NMp)
    m2g_dstp = _padi(m2g_dst, EDp, NGp - 1)

    Pg, DPg, SPg = _sc_bucket(g2m_dstp, g2m_srcp, CG, RM, 864)
    Pm, DPm, SPm = _sc_bucket(mm_dstp, mm_srcp, CM, RM, 35292)
    Pd, DPd, SPd = _sc_bucket(m2g_dstp, m2g_srcp, CG, RG, 864)

    ga16 = _sc_gather1(_padr(g2m_attr, EGp, 128), Pg, 128, 1)
    ma16 = _sc_gather1(_padr(mm_attr, EMp, 128), Pm, 128, CM // 128)
    da16 = _sc_gather1(_padr(m2g_attr, EDp, 128), Pd, 128, 1)
    ge = _embed3(ga16, p['g2m_edge_embed'], bm=512)
    me = _embed3(ma16, p['mm_edge_embed'], bm=512)
    de = _embed3(da16, p['m2g_edge_embed'], bm=512)

    gP = _proj(gx, jnp.concatenate([eWb, dWc], axis=1), bm=NGp)
    Pb_enc, Pc_dec = gP[:, :H], gP[:, H:]
    Pc_enc = _proj(mx, eWc, bm=736)

    gb = _sc_gather1(Pb_enc, SPg, H, 1)
    gc = _gather_dst_tc(Pc_enc, DPg, RM)
    ge = _edge3(ge, gb, gc, eWa, eb1, eW2, eb2, eW3, eb3, bm=512)
    agg = _seg_tc(ge, DPg, NMp, RM)
    v1a, v1b, b1, v2, b2, v3, b3 = _split_node_w(p['enc_node'])
    mx, Pb, Pc = _node3(mx, agg, v1a, b1, v1b, v2, b2, v3, b3,
                        proj=(pe[0][1], pe[0][2]), bm=736)

    for i in range(9):
        wa, _, _, b1e, w2e, b2e, w3e, b3e = pe[i]
        gb = _sc_gather1(Pb, SPm, H, CM // 128)
        gc = _gather_dst_tc(Pc, DPm, RM)
        me = _edge3(me, gb, gc, wa, b1e, w2e, b2e, w3e, b3e, bm=512)
        agg = _seg_tc(me, DPm, NMp, RM)
        v1a, v1b, b1, v2, b2, v3, b3 = pn[i]
        nxt = (pe[i + 1][1], pe[i + 1][2]) if i < 8 else (dWb, dWb)
        mx, Pb, Pc = _node3(mx, agg, v1a, b1, v1b, v2, b2, v3, b3,
                            proj=nxt, bm=736)

    gb = _sc_gather1(Pb, SPd, H, 1)
    gc = _gather_dst_tc(Pc_dec, DPd, RG)
    de = _edge3(de, gb, gc, dWa, db1, dW2, db2, dW3, db3, bm=512)
    agg = _seg_tc(de, DPd, NGp, RG)
    v1a, v1b, b1, v2, b2, v3, b3 = _split_node_w(p['dec_node'])
    gx = _node3(gx, agg, v1a, b1, v1b, v2, b2, v3, b3, proj=None, bm=NGp)

    out = _out3(gx, p['out'])
    return out[:288, :78][None]
